# baseline (speedup 1.0000x reference)
import functools

import jax
import jax.numpy as jnp
from jax import lax
from jax.experimental import pallas as pl
from jax.experimental.pallas import tpu as pltpu
from jax.experimental.pallas import tpu_sc as plsc

B = 4
N = 2048
M = 2046
MP = 2048
L = 16
NC = 2
NS = 16
NW = NC * NS
WPB = NW // B
CHUNK = N // WPB
VPW = CHUNK // L


def _mid_body(rows_ref, out_ref):
    r = rows_ref[...]
    c = r[:, 0, :] + r[:, 1, :] + r[:, 2, :]
    c1 = pltpu.roll(c, N - 1, axis=1)
    c2 = pltpu.roll(c, N - 2, axis=1)
    out_ref[...] = (c + c1 + c2) * (1.0 / 9.0)


def _compute_mid(rows):
    return pl.pallas_call(
        _mid_body,
        out_shape=jax.ShapeDtypeStruct((B, MP), jnp.float32),
    )(rows)


def _iota16():
    return lax.iota(jnp.int32, L)


def _chamfer_body(midp_hbm, ld_hbm, out_hbm, midp_v, ld_v, out_v):
    wid = lax.axis_index("c") * NS + lax.axis_index("s")
    b = wid // WPB
    base = (wid % WPB) * CHUNK

    pltpu.sync_copy(midp_hbm.at[b], midp_v)
    pltpu.sync_copy(ld_hbm.at[b], ld_v)

    iota = _iota16()

    xs = [plsc.load_gather(ld_v, [iota + (base + i * L)]) for i in range(VPW)]
    inf_v = jnp.full((L,), jnp.inf, jnp.float32)

    def body_a(jy, accs):
        yb = plsc.load_gather(midp_v, [jnp.full((L,), jy, jnp.int32)])
        out = []
        for i in range(VPW):
            d = xs[i] - yb
            out.append(jnp.minimum(accs[i], d * d))
        return tuple(out)

    accs_a = lax.fori_loop(0, M, body_a, (inf_v,) * VPW)
    s_a = accs_a[0]
    for i in range(1, VPW):
        s_a = s_a + accs_a[i]

    ys = [plsc.load_gather(midp_v, [iota + (base + i * L)]) for i in range(VPW)]

    def body_b(jx, accs):
        xb = plsc.load_gather(ld_v, [jnp.full((L,), jx, jnp.int32)])
        out = []
        for i in range(VPW):
            d = ys[i] - xb
            out.append(jnp.minimum(accs[i], d * d))
        return tuple(out)

    accs_b = lax.fori_loop(0, N, body_b, (inf_v,) * VPW)
    s_b = jnp.zeros((L,), jnp.float32)
    for i in range(VPW):
        gidx = iota + (base + i * L)
        s_b = s_b + jnp.where(gidx < M, accs_b[i], 0.0)

    out_v[...] = s_a * (1.0 / (B * N)) + s_b * (1.0 / (B * M))
    pltpu.sync_copy(out_v, out_hbm.at[wid])


def _chamfer(midp, ld):
    mesh = plsc.VectorSubcoreMesh(
        core_axis_name="c", subcore_axis_name="s", num_cores=NC, num_subcores=NS
    )
    f = pl.kernel(
        _chamfer_body,
        out_type=jax.ShapeDtypeStruct((NW, L), jnp.float32),
        mesh=mesh,
        compiler_params=pltpu.CompilerParams(needs_layout_passes=False),
        scratch_types=[
            pltpu.VMEM((MP,), jnp.float32),
            pltpu.VMEM((N,), jnp.float32),
            pltpu.VMEM((L,), jnp.float32),
        ],
    )
    return f(midp, ld)


@jax.jit
def kernel(output, lidar):
    rows = output[:, 0, 253:256, :]
    ld = lidar[:, 0, :]
    midp = _compute_mid(rows)
    parts = _chamfer(midp, ld)
    return jnp.sum(parts)


# speedup vs baseline: 1.4230x; 1.4230x over previous
import functools

import jax
import jax.numpy as jnp
from jax import lax
from jax.experimental import pallas as pl
from jax.experimental.pallas import tpu as pltpu
from jax.experimental.pallas import tpu_sc as plsc

B = 4
N = 2048
M = 2046
MP = 2048
L = 16
NC = 2
NS = 16
NW = NC * NS
WPB = NW // B

XA = 768
YA = 768
XVPW = XA // WPB // L
YVPW = YA // WPB // L
XCH = (N - XA) // 128
SCALE_X = 1.0 / (B * N)
SCALE_Y = 1.0 / (B * M)


def _mid_rows(r):
    nb = r.shape[0]
    c = r[:, 0, :] + r[:, 1, :] + r[:, 2, :]
    c1 = pltpu.roll(c, N - 1, axis=1)
    c2 = pltpu.roll(c, N - 2, axis=1)
    mid = (c + c1 + c2) * (1.0 / 9.0)
    lane = lax.broadcasted_iota(jnp.int32, (nb, MP), 1)
    return jnp.where(lane < M, mid, jnp.inf)


def _mid_body(rows_ref, out_ref):
    out_ref[...] = _mid_rows(rows_ref[...])


def _compute_mid(rows):
    return pl.pallas_call(
        _mid_body,
        out_shape=jax.ShapeDtypeStruct((B, MP), jnp.float32),
    )(rows)


def _tail_body(rows_ref, ldcol_ref, out_ref):
    midall = _mid_rows(rows_ref[...])
    lane0 = lax.broadcasted_iota(jnp.int32, (1, 128), 1) == 0
    vals = []
    for b in range(B):
        mid = midall[b:b + 1, :]
        xcol = ldcol_ref[b]

        sx = jnp.float32(0.0)
        for k in range(XCH):
            xc = xcol[XA + k * 128:XA + (k + 1) * 128, :]
            d2 = (xc - mid) ** 2
            sx = sx + jnp.sum(jnp.min(d2, axis=1))

        yr = mid[:, YA:M]
        acc = jnp.full((1, M - YA), jnp.inf, jnp.float32)
        for k in range(N // 128):
            xc = xcol[k * 128:(k + 1) * 128, :]
            d2 = (xc - yr) ** 2
            acc = jnp.minimum(acc, jnp.min(d2, axis=0, keepdims=True))
        sy = jnp.sum(acc)

        vals.append(jnp.where(lane0, sx * SCALE_X + sy * SCALE_Y, 0.0))
    out_ref[...] = jnp.concatenate(vals, axis=0)


def _tail(rows, ldcol):
    return pl.pallas_call(
        _tail_body,
        out_shape=jax.ShapeDtypeStruct((B, 128), jnp.float32),
    )(rows, ldcol)


def _iota16():
    return lax.iota(jnp.int32, L)


def _chamfer_body(midp_hbm, ld_hbm, out_hbm, midp_v, ld_v, out_v):
    wid = lax.axis_index("c") * NS + lax.axis_index("s")
    b = wid // WPB
    ci = wid % WPB

    pltpu.sync_copy(midp_hbm.at[b], midp_v)
    pltpu.sync_copy(ld_hbm.at[b], ld_v)

    iota = _iota16()
    inf_v = jnp.full((L,), jnp.inf, jnp.float32)

    xbase = ci * (XA // WPB)
    xs = [plsc.load_gather(ld_v, [iota + (xbase + i * L)]) for i in range(XVPW)]

    def body_a(jy, accs):
        yb = plsc.load_gather(midp_v, [jnp.full((L,), jy, jnp.int32)])
        out = []
        for i in range(XVPW):
            d = xs[i] - yb
            out.append(jnp.minimum(accs[i], d * d))
        return tuple(out)

    accs_a = lax.fori_loop(0, M, body_a, (inf_v,) * XVPW, unroll=2)
    s_a = accs_a[0]
    for i in range(1, XVPW):
        s_a = s_a + accs_a[i]

    ybase = ci * (YA // WPB)
    ys = [plsc.load_gather(midp_v, [iota + (ybase + i * L)]) for i in range(YVPW)]

    def body_b(jx, accs):
        xb = plsc.load_gather(ld_v, [jnp.full((L,), jx, jnp.int32)])
        out = []
        for i in range(YVPW):
            d = ys[i] - xb
            out.append(jnp.minimum(accs[i], d * d))
        return tuple(out)

    accs_b = lax.fori_loop(0, N, body_b, (inf_v,) * YVPW, unroll=2)
    s_b = accs_b[0]
    for i in range(1, YVPW):
        s_b = s_b + accs_b[i]

    out_v[...] = s_a * SCALE_X + s_b * SCALE_Y
    pltpu.sync_copy(out_v, out_hbm.at[wid])


def _chamfer_head(midp, ld):
    mesh = plsc.VectorSubcoreMesh(
        core_axis_name="c", subcore_axis_name="s", num_cores=NC, num_subcores=NS
    )
    f = pl.kernel(
        _chamfer_body,
        out_type=jax.ShapeDtypeStruct((NW, L), jnp.float32),
        mesh=mesh,
        compiler_params=pltpu.CompilerParams(needs_layout_passes=False),
        scratch_types=[
            pltpu.VMEM((MP,), jnp.float32),
            pltpu.VMEM((N,), jnp.float32),
            pltpu.VMEM((L,), jnp.float32),
        ],
    )
    return f(midp, ld)


@jax.jit
def kernel(output, lidar):
    rows = output[:, 0, 253:256, :]
    ld = lidar[:, 0, :]
    ldcol = ld[:, :, None]
    midp = _compute_mid(rows)
    head = _chamfer_head(midp, ld)
    tail = _tail(rows, ldcol)
    return jnp.sum(head) + jnp.sum(tail)


# speedup vs baseline: 1.5027x; 1.0560x over previous
import functools

import jax
import jax.numpy as jnp
from jax import lax
from jax.experimental import pallas as pl
from jax.experimental.pallas import tpu as pltpu
from jax.experimental.pallas import tpu_sc as plsc

B = 4
N = 2048
M = 2046
MP = 2048
L = 16
NC = 2
NS = 16
NW = NC * NS
WPB = NW // B

XA = 512
YA = 512
XVPW = XA // WPB // L
YVPW = YA // WPB // L
XCH = (N - XA) // 128
SCALE_X = 1.0 / (B * N)
SCALE_Y = 1.0 / (B * M)


def _mid_rows(r):
    nb = r.shape[0]
    c = r[:, 0, :] + r[:, 1, :] + r[:, 2, :]
    c1 = pltpu.roll(c, N - 1, axis=1)
    c2 = pltpu.roll(c, N - 2, axis=1)
    mid = (c + c1 + c2) * (1.0 / 9.0)
    lane = lax.broadcasted_iota(jnp.int32, (nb, MP), 1)
    return jnp.where(lane < M, mid, jnp.inf)


def _tail_body(rows_ref, ldcol_ref, out_ref):
    midall = _mid_rows(rows_ref[...])
    lane0 = lax.broadcasted_iota(jnp.int32, (1, 128), 1) == 0
    vals = []
    for b in range(B):
        mid = midall[b:b + 1, :]
        xcol = ldcol_ref[b]

        sx = jnp.float32(0.0)
        for k in range(XCH):
            xc = xcol[XA + k * 128:XA + (k + 1) * 128, :]
            d2 = (xc - mid) ** 2
            sx = sx + jnp.sum(jnp.min(d2, axis=1))

        yr = mid[:, YA:M]
        acc = jnp.full((1, M - YA), jnp.inf, jnp.float32)
        for k in range(N // 128):
            xc = xcol[k * 128:(k + 1) * 128, :]
            d2 = (xc - yr) ** 2
            acc = jnp.minimum(acc, jnp.min(d2, axis=0, keepdims=True))
        sy = jnp.sum(acc)

        vals.append(jnp.where(lane0, sx * SCALE_X + sy * SCALE_Y, 0.0))
    out_ref[...] = jnp.concatenate(vals, axis=0)


def _tail(rows, ldcol):
    return pl.pallas_call(
        _tail_body,
        out_shape=jax.ShapeDtypeStruct((B, 128), jnp.float32),
    )(rows, ldcol)


def _iota16():
    return lax.iota(jnp.int32, L)


def _chamfer_body(rowsf_hbm, ld_hbm, out_hbm, rowsf_v, c_v, midp_v, ld_v, out_v):
    wid = lax.axis_index("c") * NS + lax.axis_index("s")
    b = wid // WPB
    ci = wid % WPB

    pltpu.sync_copy(rowsf_hbm.at[b], rowsf_v)
    pltpu.sync_copy(ld_hbm.at[b], ld_v)

    iota = _iota16()
    inf_v = jnp.full((L,), jnp.inf, jnp.float32)

    def mid_step(i, _):
        idx = iota + i * L
        c = (plsc.load_gather(rowsf_v, [idx])
             + plsc.load_gather(rowsf_v, [idx + N])
             + plsc.load_gather(rowsf_v, [idx + 2 * N]))
        plsc.store_scatter(c_v, [idx], c)
        return 0

    lax.fori_loop(0, MP // L, mid_step, 0)

    def mid_step2(i, _):
        idx = iota + i * L
        i1 = jnp.minimum(idx + 1, N - 1)
        i2 = jnp.minimum(idx + 2, N - 1)
        m = (plsc.load_gather(c_v, [idx])
             + plsc.load_gather(c_v, [i1])
             + plsc.load_gather(c_v, [i2])) * (1.0 / 9.0)
        m = jnp.where(idx < M, m, jnp.inf)
        plsc.store_scatter(midp_v, [idx], m)
        return 0

    lax.fori_loop(0, MP // L, mid_step2, 0)

    xbase = ci * (XA // WPB)
    xs = [plsc.load_gather(ld_v, [iota + (xbase + i * L)]) for i in range(XVPW)]

    def body_a(jy, accs):
        yb = plsc.load_gather(midp_v, [jnp.full((L,), jy, jnp.int32)])
        out = []
        for i in range(XVPW):
            d = xs[i] - yb
            out.append(jnp.minimum(accs[i], d * d))
        return tuple(out)

    accs_a = lax.fori_loop(0, M, body_a, (inf_v,) * XVPW, unroll=2)
    s_a = accs_a[0]
    for i in range(1, XVPW):
        s_a = s_a + accs_a[i]

    ybase = ci * (YA // WPB)
    ys = [plsc.load_gather(midp_v, [iota + (ybase + i * L)]) for i in range(YVPW)]

    def body_b(jx, accs):
        xb = plsc.load_gather(ld_v, [jnp.full((L,), jx, jnp.int32)])
        out = []
        for i in range(YVPW):
            d = ys[i] - xb
            out.append(jnp.minimum(accs[i], d * d))
        return tuple(out)

    accs_b = lax.fori_loop(0, N, body_b, (inf_v,) * YVPW, unroll=2)
    s_b = accs_b[0]
    for i in range(1, YVPW):
        s_b = s_b + accs_b[i]

    out_v[...] = s_a * SCALE_X + s_b * SCALE_Y
    pltpu.sync_copy(out_v, out_hbm.at[wid])


def _chamfer_head(rowsf, ld):
    mesh = plsc.VectorSubcoreMesh(
        core_axis_name="c", subcore_axis_name="s", num_cores=NC, num_subcores=NS
    )
    f = pl.kernel(
        _chamfer_body,
        out_type=jax.ShapeDtypeStruct((NW, L), jnp.float32),
        mesh=mesh,
        compiler_params=pltpu.CompilerParams(needs_layout_passes=False),
        scratch_types=[
            pltpu.VMEM((3 * N,), jnp.float32),
            pltpu.VMEM((N,), jnp.float32),
            pltpu.VMEM((MP,), jnp.float32),
            pltpu.VMEM((N,), jnp.float32),
            pltpu.VMEM((L,), jnp.float32),
        ],
    )
    return f(rowsf, ld)


@jax.jit
def kernel(output, lidar):
    rows = output[:, 0, 253:256, :]
    ld = lidar[:, 0, :]
    ldcol = ld[:, :, None]
    rowsf = rows.reshape(B, 3 * N)
    head = _chamfer_head(rowsf, ld)
    tail = _tail(rows, ldcol)
    return jnp.sum(head) + jnp.sum(tail)


# speedup vs baseline: 1.6370x; 1.0894x over previous
import functools

import jax
import jax.numpy as jnp
from jax import lax
from jax.experimental import pallas as pl
from jax.experimental.pallas import tpu as pltpu
from jax.experimental.pallas import tpu_sc as plsc

B = 4
H = 256
N = 2048
M = 2046
MP = 2048
L = 16
NC = 2
NS = 16
NW = NC * NS
WPB = NW // B

XA = 512
YA = 512
XVPW = XA // WPB // L
YVPW = YA // WPB // L
SCALE_X = 1.0 / (B * N)
SCALE_Y = 1.0 / (B * M)


def _tail_body(rows_ref, ldr_ref, out_ref):
    r = rows_ref[0, 0]
    c = r[5:6, :] + r[6:7, :] + r[7:8, :]
    c1 = pltpu.roll(c, N - 1, axis=1)
    c2 = pltpu.roll(c, N - 2, axis=1)
    mid = (c + c1 + c2) * (1.0 / 9.0)
    lane = lax.broadcasted_iota(jnp.int32, (1, MP), 1)
    mid = jnp.where(lane < M, mid, jnp.inf)

    ldT = jnp.transpose(ldr_ref[0], (1, 0))
    inf128 = jnp.full((128, 128), jnp.inf, jnp.float32)
    laneid = lax.broadcasted_iota(jnp.int32, (1, 128), 1)

    sx = jnp.float32(0.0)
    for k in range(XA // 128, N // 128):
        xc = ldT[:, k:k + 1]
        acc = inf128
        for j in range(MP // 128):
            yb = mid[:, j * 128:(j + 1) * 128]
            acc = jnp.minimum(acc, (xc - yb) ** 2)
        sx = sx + jnp.sum(jnp.min(acc, axis=1))

    sy = jnp.float32(0.0)
    for j in range(YA // 128, MP // 128):
        yb = mid[:, j * 128:(j + 1) * 128]
        acc = inf128
        for k in range(N // 128):
            xc = ldT[:, k:k + 1]
            acc = jnp.minimum(acc, (xc - yb) ** 2)
        v = jnp.min(acc, axis=0, keepdims=True)
        v = jnp.where(laneid + j * 128 < M, v, 0.0)
        sy = sy + jnp.sum(v)

    val = sx * SCALE_X + sy * SCALE_Y
    subl = lax.broadcasted_iota(jnp.int32, (8, 128), 0)
    lane8 = lax.broadcasted_iota(jnp.int32, (8, 128), 1)
    out_ref[...] = jnp.where((subl == 0) & (lane8 == 0), val, 0.0)


def _tail(output, ldr):
    return pl.pallas_call(
        _tail_body,
        grid=(B,),
        in_specs=[
            pl.BlockSpec((1, 1, 8, N), lambda b: (b, 0, (H // 8) - 1, 0)),
            pl.BlockSpec((1, N // 128, 128), lambda b: (b, 0, 0)),
        ],
        out_specs=pl.BlockSpec((8, 128), lambda b: (b, 0)),
        out_shape=jax.ShapeDtypeStruct((B * 8, 128), jnp.float32),
    )(output, ldr)


def _iota16():
    return lax.iota(jnp.int32, L)


_GDN = lax.GatherDimensionNumbers(
    offset_dims=(), collapsed_slice_dims=(0,), start_index_map=(0,)
)


def _bcast_lane(v, l):
    idx = jnp.full((L, 1), l, jnp.int32)
    return lax.gather(v, idx, _GDN, (1,),
                      mode=lax.GatherScatterMode.PROMISE_IN_BOUNDS)


def _chamfer_body(out4_hbm, lidar_hbm, out_hbm, rows_v, c_v, midp_v, ld_v, out_v):
    wid = lax.axis_index("c") * NS + lax.axis_index("s")
    b = wid // WPB
    ci = wid % WPB

    pltpu.sync_copy(out4_hbm.at[b, 0, pl.ds(H - 3, 3)], rows_v)
    pltpu.sync_copy(lidar_hbm.at[b, 0], ld_v)

    iota = _iota16()
    inf_v = jnp.full((L,), jnp.inf, jnp.float32)

    def mid_step(i, _):
        idx = iota + i * L
        c = (plsc.load_gather(rows_v, [jnp.full((L,), 0, jnp.int32), idx])
             + plsc.load_gather(rows_v, [jnp.full((L,), 1, jnp.int32), idx])
             + plsc.load_gather(rows_v, [jnp.full((L,), 2, jnp.int32), idx]))
        plsc.store_scatter(c_v, [idx], c)
        return 0

    lax.fori_loop(0, MP // L, mid_step, 0)

    def mid_step2(i, _):
        idx = iota + i * L
        i1 = jnp.minimum(idx + 1, N - 1)
        i2 = jnp.minimum(idx + 2, N - 1)
        m = (plsc.load_gather(c_v, [idx])
             + plsc.load_gather(c_v, [i1])
             + plsc.load_gather(c_v, [i2])) * (1.0 / 9.0)
        m = jnp.where(idx < M, m, jnp.inf)
        plsc.store_scatter(midp_v, [idx], m)
        return 0

    lax.fori_loop(0, MP // L, mid_step2, 0)

    xbase = ci * (XA // WPB)
    xs = [plsc.load_gather(ld_v, [iota + (xbase + i * L)]) for i in range(XVPW)]

    def body_a(jc, accs):
        ych = plsc.load_gather(midp_v, [iota + jc * L])
        acc = list(accs)
        for l in range(L):
            yb = _bcast_lane(ych, l)
            for i in range(XVPW):
                d = xs[i] - yb
                acc[i] = jnp.minimum(acc[i], d * d)
        return tuple(acc)

    accs_a = lax.fori_loop(0, MP // L, body_a, (inf_v,) * XVPW)
    s_a = accs_a[0]
    for i in range(1, XVPW):
        s_a = s_a + accs_a[i]

    ybase = ci * (YA // WPB)
    ys = [plsc.load_gather(midp_v, [iota + (ybase + i * L)]) for i in range(YVPW)]

    def body_b(jc, accs):
        xch = plsc.load_gather(ld_v, [iota + jc * L])
        acc = list(accs)
        for l in range(L):
            xb = _bcast_lane(xch, l)
            for i in range(YVPW):
                d = ys[i] - xb
                acc[i] = jnp.minimum(acc[i], d * d)
        return tuple(acc)

    accs_b = lax.fori_loop(0, N // L, body_b, (inf_v,) * YVPW)
    s_b = accs_b[0]
    for i in range(1, YVPW):
        s_b = s_b + accs_b[i]

    out_v[...] = s_a * SCALE_X + s_b * SCALE_Y
    pltpu.sync_copy(out_v, out_hbm.at[wid])


def _chamfer_head(output, lidar):
    mesh = plsc.VectorSubcoreMesh(
        core_axis_name="c", subcore_axis_name="s", num_cores=NC, num_subcores=NS
    )
    f = pl.kernel(
        _chamfer_body,
        out_type=jax.ShapeDtypeStruct((NW, L), jnp.float32),
        mesh=mesh,
        compiler_params=pltpu.CompilerParams(needs_layout_passes=False),
        scratch_types=[
            pltpu.VMEM((3, N), jnp.float32),
            pltpu.VMEM((N,), jnp.float32),
            pltpu.VMEM((MP,), jnp.float32),
            pltpu.VMEM((N,), jnp.float32),
            pltpu.VMEM((L,), jnp.float32),
        ],
    )
    return f(output, lidar)


@jax.jit
def kernel(output, lidar):
    ldr = lidar.reshape(B, N // 128, 128)
    head = _chamfer_head(output, lidar)
    tail = _tail(output, ldr)
    return jnp.sum(head) + jnp.sum(tail)


# speedup vs baseline: 1.7481x; 1.0679x over previous
import functools

import jax
import jax.numpy as jnp
from jax import lax
from jax.experimental import pallas as pl
from jax.experimental.pallas import tpu as pltpu
from jax.experimental.pallas import tpu_sc as plsc

B = 4
H = 256
N = 2048
M = 2046
MP = 2048
L = 16
NC = 2
NS = 16
NW = NC * NS
WPB = NW // B

XA = 256
YA = 256
XVPW = XA // WPB // L
YVPW = YA // WPB // L
SCALE_X = 1.0 / (B * N)
SCALE_Y = 1.0 / (B * M)


def _tail_body(rows_ref, ldr_ref, out_ref):
    r = rows_ref[0, 0]
    c = r[5:6, :] + r[6:7, :] + r[7:8, :]
    c1 = pltpu.roll(c, N - 1, axis=1)
    c2 = pltpu.roll(c, N - 2, axis=1)
    mid = (c + c1 + c2) * (1.0 / 9.0)
    lane = lax.broadcasted_iota(jnp.int32, (1, MP), 1)
    mid = jnp.where(lane < M, mid, jnp.inf)

    ldT = jnp.transpose(ldr_ref[0], (1, 0))
    inf128 = jnp.full((128, 128), jnp.inf, jnp.float32)
    laneid = lax.broadcasted_iota(jnp.int32, (1, 128), 1)

    sx = jnp.float32(0.0)
    for k in range(XA // 128, N // 128):
        xc = ldT[:, k:k + 1]
        acc = inf128
        for j in range(MP // 128):
            yb = mid[:, j * 128:(j + 1) * 128]
            acc = jnp.minimum(acc, (xc - yb) ** 2)
        sx = sx + jnp.sum(jnp.min(acc, axis=1))

    sy = jnp.float32(0.0)
    for j in range(YA // 128, MP // 128):
        yb = mid[:, j * 128:(j + 1) * 128]
        acc = inf128
        for k in range(N // 128):
            xc = ldT[:, k:k + 1]
            acc = jnp.minimum(acc, (xc - yb) ** 2)
        v = jnp.min(acc, axis=0, keepdims=True)
        v = jnp.where(laneid + j * 128 < M, v, 0.0)
        sy = sy + jnp.sum(v)

    val = sx * SCALE_X + sy * SCALE_Y
    subl = lax.broadcasted_iota(jnp.int32, (8, 128), 0)
    lane8 = lax.broadcasted_iota(jnp.int32, (8, 128), 1)
    out_ref[...] = jnp.where((subl == 0) & (lane8 == 0), val, 0.0)


def _tail(output, ldr):
    return pl.pallas_call(
        _tail_body,
        grid=(B,),
        in_specs=[
            pl.BlockSpec((1, 1, 8, N), lambda b: (b, 0, (H // 8) - 1, 0)),
            pl.BlockSpec((1, N // 128, 128), lambda b: (b, 0, 0)),
        ],
        out_specs=pl.BlockSpec((8, 128), lambda b: (b, 0)),
        out_shape=jax.ShapeDtypeStruct((B * 8, 128), jnp.float32),
    )(output, ldr)


def _iota16():
    return lax.iota(jnp.int32, L)


_GDN = lax.GatherDimensionNumbers(
    offset_dims=(), collapsed_slice_dims=(0,), start_index_map=(0,)
)


def _bcast_lane(v, l):
    idx = jnp.full((L, 1), l, jnp.int32)
    return lax.gather(v, idx, _GDN, (1,),
                      mode=lax.GatherScatterMode.PROMISE_IN_BOUNDS)


def _chamfer_body(out4_hbm, lidar_hbm, out_hbm, rows_v, c_v, midp_v, ld_v, out_v):
    wid = lax.axis_index("c") * NS + lax.axis_index("s")
    b = wid // WPB
    ci = wid % WPB

    pltpu.sync_copy(out4_hbm.at[b, 0, pl.ds(H - 3, 3)], rows_v)
    pltpu.sync_copy(lidar_hbm.at[b, 0], ld_v)

    iota = _iota16()
    inf_v = jnp.full((L,), jnp.inf, jnp.float32)

    def mid_step(i, _):
        idx = iota + i * L
        c = (plsc.load_gather(rows_v, [jnp.full((L,), 0, jnp.int32), idx])
             + plsc.load_gather(rows_v, [jnp.full((L,), 1, jnp.int32), idx])
             + plsc.load_gather(rows_v, [jnp.full((L,), 2, jnp.int32), idx]))
        plsc.store_scatter(c_v, [idx], c)
        return 0

    lax.fori_loop(0, MP // L, mid_step, 0)

    def mid_step2(i, _):
        idx = iota + i * L
        i1 = jnp.minimum(idx + 1, N - 1)
        i2 = jnp.minimum(idx + 2, N - 1)
        m = (plsc.load_gather(c_v, [idx])
             + plsc.load_gather(c_v, [i1])
             + plsc.load_gather(c_v, [i2])) * (1.0 / 9.0)
        m = jnp.where(idx < M, m, jnp.inf)
        plsc.store_scatter(midp_v, [idx], m)
        return 0

    lax.fori_loop(0, MP // L, mid_step2, 0)

    xbase = ci * (XA // WPB)
    xs = [plsc.load_gather(ld_v, [iota + (xbase + i * L)]) for i in range(XVPW)]

    def body_a(jc, accs):
        ych = plsc.load_gather(midp_v, [iota + jc * L])
        acc = list(accs)
        for l in range(L):
            yb = _bcast_lane(ych, l)
            for i in range(XVPW):
                d = xs[i] - yb
                acc[i] = jnp.minimum(acc[i], d * d)
        return tuple(acc)

    accs_a = lax.fori_loop(0, MP // L, body_a, (inf_v,) * XVPW)
    s_a = accs_a[0]
    for i in range(1, XVPW):
        s_a = s_a + accs_a[i]

    ybase = ci * (YA // WPB)
    ys = [plsc.load_gather(midp_v, [iota + (ybase + i * L)]) for i in range(YVPW)]

    def body_b(jc, accs):
        xch = plsc.load_gather(ld_v, [iota + jc * L])
        acc = list(accs)
        for l in range(L):
            xb = _bcast_lane(xch, l)
            for i in range(YVPW):
                d = ys[i] - xb
                acc[i] = jnp.minimum(acc[i], d * d)
        return tuple(acc)

    accs_b = lax.fori_loop(0, N // L, body_b, (inf_v,) * YVPW)
    s_b = accs_b[0]
    for i in range(1, YVPW):
        s_b = s_b + accs_b[i]

    out_v[...] = s_a * SCALE_X + s_b * SCALE_Y
    pltpu.sync_copy(out_v, out_hbm.at[wid])


def _chamfer_head(output, lidar):
    mesh = plsc.VectorSubcoreMesh(
        core_axis_name="c", subcore_axis_name="s", num_cores=NC, num_subcores=NS
    )
    f = pl.kernel(
        _chamfer_body,
        out_type=jax.ShapeDtypeStruct((NW, L), jnp.float32),
        mesh=mesh,
        compiler_params=pltpu.CompilerParams(needs_layout_passes=False),
        scratch_types=[
            pltpu.VMEM((3, N), jnp.float32),
            pltpu.VMEM((N,), jnp.float32),
            pltpu.VMEM((MP,), jnp.float32),
            pltpu.VMEM((N,), jnp.float32),
            pltpu.VMEM((L,), jnp.float32),
        ],
    )
    return f(output, lidar)


@jax.jit
def kernel(output, lidar):
    ldr = lidar.reshape(B, N // 128, 128)
    head = _chamfer_head(output, lidar)
    tail = _tail(output, ldr)
    return jnp.sum(head) + jnp.sum(tail)
